# baseline (speedup 1.0000x reference)
import functools

import jax
import jax.numpy as jnp
from jax import lax
from jax.experimental import pallas as pl
from jax.experimental.pallas import tpu as pltpu
from jax.experimental.pallas import tpu_sc as plsc

N = 10000
E = 320000
D = 128
NS = 16
K = 128
EPT = 20480
NCHUNK = EPT // K
SUP = 16
NOUT = NCHUNK // SUP
EPAD = NS * EPT
NP = 10240
RPW = NP // NS

_mesh = plsc.VectorSubcoreMesh(core_axis_name="c", subcore_axis_name="s",
                               num_cores=2, num_subcores=NS)


def _make_sc_scatter(with_counts: bool):
    out_type = [
        jax.ShapeDtypeStruct((2, NP, D), jnp.float32),
    ]
    if with_counts:
        out_type += [
            jax.ShapeDtypeStruct((2, NP, D), jnp.float32),
        ]
    scratch = [
        pltpu.VMEM_SHARED((NP, D), jnp.float32),
        pltpu.VMEM((SUP, K), jnp.int32),
        pltpu.VMEM((SUP, K), jnp.int32),
        pltpu.VMEM((K, D), jnp.float32),
        pltpu.VMEM((K, D), jnp.float32),
        pltpu.SemaphoreType.DMA,
        pltpu.SemaphoreType.DMA,
        pltpu.SemaphoreType.DMA,
        pltpu.SemaphoreType.DMA,
    ]

    @functools.partial(pl.kernel, out_type=out_type, mesh=_mesh,
                       scratch_types=scratch)
    def sc_kernel(table, rows_all, cols_all, z128, ones_h, *rest):
        if with_counts:
            sums, counts, acc, idx_row, idx_col, bufa, bufb, gsa, gsb, ssa, ssb = rest
        else:
            sums, acc, idx_row, idx_col, bufa, bufb, gsa, gsb, ssa, ssb = rest
        bufs = (bufa, bufb)
        gsems = (gsa, gsb)
        ssems = (ssa, ssb)

        c = lax.axis_index("c")
        s = lax.axis_index("s")

        def tile_slices():
            return [pl.ds(s * RPW + i * K, K) for i in range(RPW // K)]

        def zero_acc():
            pltpu.sync_copy(z128, bufa)
            for sli in tile_slices():
                pltpu.sync_copy(bufa, acc.at[sli])

        def publish(dst):
            for sli in tile_slices():
                pltpu.sync_copy(acc.at[sli], bufa)
                pltpu.sync_copy(bufa, dst.at[c, sli])

        def stage_idx(jo, cols=True):
            pltpu.sync_copy(rows_all.at[c, s * NOUT + jo], idx_row)
            if cols:
                pltpu.sync_copy(cols_all.at[c, s * NOUT + jo], idx_col)

        zero_acc()

        if with_counts:
            pltpu.sync_copy(ones_h, bufb)
            plsc.subcore_barrier()

            def outer0(jo, carry):
                stage_idx(jo, cols=False)

                def inner0(j, carry2):
                    pltpu.sync_copy(bufb, acc.at[idx_row.at[j]], add=True)
                    return carry2

                lax.fori_loop(0, SUP, inner0, 0)
                return carry

            lax.fori_loop(0, NOUT, outer0, 0)
            plsc.subcore_barrier()
            publish(counts)
            zero_acc()

        plsc.subcore_barrier()

        def outer(jo, carry):
            stage_idx(jo)

            def inner(j, carry2):
                pltpu.async_copy(table.at[idx_col.at[j]], bufa, gsa).wait()
                pltpu.sync_copy(bufa, acc.at[idx_row.at[j]], add=True)
                return carry2

            lax.fori_loop(0, SUP, inner, 0)
            return carry

        lax.fori_loop(0, NOUT, outer, 0)

        plsc.subcore_barrier()
        publish(sums)

    return sc_kernel


_sc_phase1 = _make_sc_scatter(with_counts=True)
_sc_phase2 = _make_sc_scatter(with_counts=False)

R = 1000


def _ln_gelu(h, g, b):
    mu = jnp.mean(h, axis=-1, keepdims=True)
    var = jnp.mean((h - mu) ** 2, axis=-1, keepdims=True)
    y = (h - mu) * lax.rsqrt(var + 1e-5) * g + b
    return 0.5 * y * (1.0 + lax.erf(y * 0.7071067811865476))


def _sage_post(x, nei, WsT, WnT, b0, g, b):
    h = (jnp.dot(x, WsT, preferred_element_type=jnp.float32)
         + jnp.dot(nei, WnT, preferred_element_type=jnp.float32) + b0)
    return _ln_gelu(h, g, b)


def _tc_a_body(x_ref, ss_ref, cs_ref, sa_ref, ca_ref,
               WsT_ref, WnT_ref, b0s_ref, gs_ref, bs_ref,
               WaT_ref, WanT_ref, b0a_ref, ga_ref, ba_ref, out_ref):
    x = x_ref[...]
    nei_s = ss_ref[...] / (cs_ref[:, 0:1] + 1e-12)
    nei_a = sa_ref[...] / (ca_ref[:, 0:1] + 1e-12)
    out_ref[0] = _sage_post(x, nei_s, WsT_ref[...], WnT_ref[...],
                            b0s_ref[...], gs_ref[...], bs_ref[...])
    out_ref[1] = _sage_post(x, nei_a, WaT_ref[...], WanT_ref[...],
                            b0a_ref[...], ga_ref[...], ba_ref[...])


def _tc_a(x, sum_s, cnt_s, sum_a, cnt_a, *weights):
    blk = lambda shp: pl.BlockSpec(shp, lambda i: (i, 0))
    full = lambda a: pl.BlockSpec(a.shape, lambda i: (0,) * a.ndim)
    return pl.pallas_call(
        _tc_a_body,
        grid=(N // R,),
        in_specs=[blk((R, D)), blk((R, D)), blk((R, D)), blk((R, D)),
                  blk((R, D))] + [full(w) for w in weights],
        out_specs=pl.BlockSpec((2, R, D), lambda i: (0, i, 0)),
        out_shape=jax.ShapeDtypeStruct((2, N, D), jnp.float32),
    )(x, sum_s, cnt_s, sum_a, cnt_a, *weights)


def _tc_b_body(h1_ref, ss_ref, cs_ref, sa_ref, ca_ref,
               WsT_ref, WnT_ref, b0s_ref, gs_ref, bs_ref,
               WaT_ref, WanT_ref, b0a_ref, ga_ref, ba_ref,
               ncg_ref, ncb_ref, fg1t_ref, fg1b_ref, fg2w_ref, fg2b_ref,
               rpt_ref, rpb_ref, out_ref):
    hs1 = h1_ref[0]
    ha1 = h1_ref[1]
    nei_s = ss_ref[...] / (cs_ref[:, 0:1] + 1e-12)
    nei_a = sa_ref[...] / (ca_ref[:, 0:1] + 1e-12)
    hs2 = _sage_post(hs1, nei_s, WsT_ref[...], WnT_ref[...],
                     b0s_ref[...], gs_ref[...], bs_ref[...])
    ha2 = _sage_post(ha1, nei_a, WaT_ref[...], WanT_ref[...],
                     b0a_ref[...], ga_ref[...], ba_ref[...])

    pieces = (hs1, hs2, ha1, ha2)
    tot = sum(jnp.sum(p, axis=-1, keepdims=True) for p in pieces)
    totq = sum(jnp.sum(p * p, axis=-1, keepdims=True) for p in pieces)
    mu = tot * (1.0 / 512.0)
    var = totq * (1.0 / 512.0) - mu * mu
    rstd = lax.rsqrt(var + 1e-5)
    ncg = ncg_ref[...]
    ncb = ncb_ref[...]
    fg1t = fg1t_ref[...]
    acc = fg1b_ref[...]
    for i, p in enumerate(pieces):
        cc = (p - mu) * rstd * ncg[i] + ncb[i]
        acc = acc + jnp.dot(cc, fg1t[i * D:(i + 1) * D],
                            preferred_element_type=jnp.float32)
    g1 = jnp.maximum(acc, 0.0)
    w = jax.nn.sigmoid(jnp.sum(g1 * fg2w_ref[...], axis=-1, keepdims=True)
                       + fg2b_ref[0, 0])
    f1 = w * hs1 + (1.0 - w) * ha1
    f2 = w * hs2 + (1.0 - w) * ha2
    rpt = rpt_ref[...]
    out_ref[...] = (jnp.dot(f1, rpt[0:D], preferred_element_type=jnp.float32)
                    + jnp.dot(f2, rpt[D:2 * D],
                              preferred_element_type=jnp.float32)
                    + rpb_ref[...])


def _tc_b(h1, sum_s, cnt_s, sum_a, cnt_a, *weights):
    blk = lambda shp: pl.BlockSpec(shp, lambda i: (i, 0))
    full = lambda a: pl.BlockSpec(a.shape, lambda i: (0,) * a.ndim)
    return pl.pallas_call(
        _tc_b_body,
        grid=(N // R,),
        in_specs=[pl.BlockSpec((2, R, D), lambda i: (0, i, 0)),
                  blk((R, D)), blk((R, D)), blk((R, D)), blk((R, D))]
                 + [full(w) for w in weights],
        out_specs=blk((R, D)),
        out_shape=jax.ShapeDtypeStruct((N, D), jnp.float32),
    )(h1, sum_s, cnt_s, sum_a, cnt_a, *weights)


def kernel(x, edge_spatial, edge_attr,
           s0_Wself, s0_bself, s0_Wnei, s0_bnei, s0_g, s0_b,
           s1_Wself, s1_bself, s1_Wnei, s1_bnei, s1_g, s1_b,
           a0_Wself, a0_bself, a0_Wnei, a0_bnei, a0_g, a0_b,
           a1_Wself, a1_bself, a1_Wnei, a1_bnei, a1_g, a1_b,
           nc_g, nc_b, fg1_W, fg1_b, fg2_W, fg2_b, rp_W, rp_b):
    f32 = jnp.float32
    pad_r = jnp.full((EPAD - E,), NP - 8, jnp.int32)
    pad_c = jnp.zeros((EPAD - E,), jnp.int32)
    pad = lambda v, p: jnp.concatenate([v, p])
    idx4 = lambda a, b: jnp.stack([a, b]).reshape(2, NS * NOUT, SUP, K)
    rows_all = idx4(pad(edge_spatial[0], pad_r), pad(edge_attr[0], pad_r))
    cols1 = idx4(pad(edge_spatial[1], pad_c), pad(edge_attr[1], pad_c))
    cols2 = idx4(pad(edge_spatial[1], pad_c), pad(edge_attr[1] + N, pad_c))
    z128 = jnp.zeros((K, D), f32)
    ones_h = jnp.ones((K, D), f32)

    sums0, counts = _sc_phase1(x, rows_all, cols1, z128, ones_h)
    sum_s0, sum_a0 = sums0[0], sums0[1]
    cnt_s, cnt_a = counts[0], counts[1]

    row = lambda v: v.reshape(1, -1)
    wa = (s0_Wself.T, s0_Wnei.T, row(s0_bself + s0_bnei), row(s0_g),
          row(s0_b), a0_Wself.T, a0_Wnei.T, row(a0_bself + a0_bnei),
          row(a0_g), row(a0_b))
    h1 = _tc_a(x, sum_s0, cnt_s, sum_a0, cnt_a, *wa)

    sums1, = _sc_phase2(h1.reshape(2 * N, D), rows_all, cols2, z128, ones_h)
    sum_s1, sum_a1 = sums1[0], sums1[1]

    wb = (s1_Wself.T, s1_Wnei.T, row(s1_bself + s1_bnei), row(s1_g),
          row(s1_b), a1_Wself.T, a1_Wnei.T, row(a1_bself + a1_bnei),
          row(a1_g), row(a1_b),
          nc_g.reshape(4, D), nc_b.reshape(4, D),
          fg1_W.T, row(fg1_b), fg2_W, fg2_b.reshape(1, 1),
          rp_W.T, rp_b.reshape(1, D))
    return _tc_b(h1, sum_s1, cnt_s, sum_a1, cnt_a, *wb)


# speedup vs baseline: 2.3602x; 2.3602x over previous
import functools

import jax
import jax.numpy as jnp
from jax import lax
from jax.experimental import pallas as pl
from jax.experimental.pallas import tpu as pltpu
from jax.experimental.pallas import tpu_sc as plsc

N = 10000
E = 320000
D = 128
NS = 16
K = 80
EPT = 20000
NCHUNK = EPT // K
SUP = 10
NOUT = NCHUNK // SUP
EPAD = NS * EPT
NP = 10240
RPW = NP // NS

_mesh = plsc.VectorSubcoreMesh(core_axis_name="c", subcore_axis_name="s",
                               num_cores=2, num_subcores=NS)


def _make_sc_scatter(with_counts: bool):
    out_type = [
        jax.ShapeDtypeStruct((2, NP, D), jnp.float32),
    ]
    if with_counts:
        out_type += [
            jax.ShapeDtypeStruct((2, NP, D), jnp.float32),
        ]
    scratch = [
        pltpu.VMEM_SHARED((NP, D), jnp.float32),
        pltpu.VMEM((SUP, K), jnp.int32),
        pltpu.VMEM((SUP, K), jnp.int32),
        pltpu.VMEM((K, D), jnp.float32),
        pltpu.VMEM((K, D), jnp.float32),
        pltpu.SemaphoreType.DMA,
        pltpu.SemaphoreType.DMA,
        pltpu.SemaphoreType.DMA,
        pltpu.SemaphoreType.DMA,
    ]

    @functools.partial(pl.kernel, out_type=out_type, mesh=_mesh,
                       scratch_types=scratch)
    def sc_kernel(table, rows_all, cols_all, z128, ones_h, *rest):
        if with_counts:
            sums, counts, acc, idx_row, idx_col, bufa, bufb, gsa, gsb, ssa, ssb = rest
        else:
            sums, acc, idx_row, idx_col, bufa, bufb, gsa, gsb, ssa, ssb = rest
        bufs = (bufa, bufb)
        gsems = (gsa, gsb)
        ssems = (ssa, ssb)

        c = lax.axis_index("c")
        s = lax.axis_index("s")

        def tile_slices():
            return [pl.ds(s * RPW + i * K, K) for i in range(RPW // K)]

        def zero_acc():
            pltpu.sync_copy(z128, bufa)
            for sli in tile_slices():
                pltpu.sync_copy(bufa, acc.at[sli])

        def publish(dst):
            for sli in tile_slices():
                pltpu.sync_copy(acc.at[sli], bufa)
                pltpu.sync_copy(bufa, dst.at[c, sli])

        def stage_idx(jo, cols=True):
            pltpu.sync_copy(rows_all.at[c, s * NOUT + jo], idx_row)
            if cols:
                pltpu.sync_copy(cols_all.at[c, s * NOUT + jo], idx_col)

        zero_acc()

        if with_counts:
            pltpu.sync_copy(ones_h, bufb)
            plsc.subcore_barrier()

            def outer0(jo, carry):
                stage_idx(jo, cols=False)

                def inner0(j, carry2):
                    pltpu.sync_copy(bufb, acc.at[idx_row.at[j]], add=True)
                    return carry2

                lax.fori_loop(0, SUP, inner0, 0)
                return carry

            lax.fori_loop(0, NOUT, outer0, 0)
            plsc.subcore_barrier()
            publish(counts)
            zero_acc()

        plsc.subcore_barrier()

        def outer(jo, carry):
            stage_idx(jo)
            gcp = [None] * SUP
            scp = [None] * SUP
            gcp[0] = pltpu.async_copy(table.at[idx_col.at[0]], bufs[0],
                                      gsems[0])
            for j in range(SUP):
                b = j % 2
                if j + 1 < SUP:
                    if j >= 1:
                        scp[j - 1].wait()
                    gcp[j + 1] = pltpu.async_copy(
                        table.at[idx_col.at[j + 1]], bufs[1 - b],
                        gsems[1 - b])
                gcp[j].wait()
                scp[j] = pltpu.async_copy(bufs[b], acc.at[idx_row.at[j]],
                                          ssems[b], add=True)
            scp[SUP - 2].wait()
            scp[SUP - 1].wait()
            return carry

        lax.fori_loop(0, NOUT, outer, 0)

        plsc.subcore_barrier()
        publish(sums)

    return sc_kernel


_sc_phase1 = _make_sc_scatter(with_counts=True)
_sc_phase2 = _make_sc_scatter(with_counts=False)

R = 1000


def _ln_gelu(h, g, b):
    mu = jnp.mean(h, axis=-1, keepdims=True)
    var = jnp.mean((h - mu) ** 2, axis=-1, keepdims=True)
    y = (h - mu) * lax.rsqrt(var + 1e-5) * g + b
    return 0.5 * y * (1.0 + lax.erf(y * 0.7071067811865476))


def _sage_post(x, nei, WsT, WnT, b0, g, b):
    h = (jnp.dot(x, WsT, preferred_element_type=jnp.float32)
         + jnp.dot(nei, WnT, preferred_element_type=jnp.float32) + b0)
    return _ln_gelu(h, g, b)


def _tc_a_body(x_ref, ss_ref, cs_ref, sa_ref, ca_ref,
               WsT_ref, WnT_ref, b0s_ref, gs_ref, bs_ref,
               WaT_ref, WanT_ref, b0a_ref, ga_ref, ba_ref, out_ref):
    x = x_ref[...]
    nei_s = ss_ref[...] / (cs_ref[:, 0:1] + 1e-12)
    nei_a = sa_ref[...] / (ca_ref[:, 0:1] + 1e-12)
    out_ref[0] = _sage_post(x, nei_s, WsT_ref[...], WnT_ref[...],
                            b0s_ref[...], gs_ref[...], bs_ref[...])
    out_ref[1] = _sage_post(x, nei_a, WaT_ref[...], WanT_ref[...],
                            b0a_ref[...], ga_ref[...], ba_ref[...])


def _tc_a(x, sum_s, cnt_s, sum_a, cnt_a, *weights):
    blk = lambda shp: pl.BlockSpec(shp, lambda i: (i, 0))
    full = lambda a: pl.BlockSpec(a.shape, lambda i: (0,) * a.ndim)
    return pl.pallas_call(
        _tc_a_body,
        grid=(N // R,),
        in_specs=[blk((R, D)), blk((R, D)), blk((R, D)), blk((R, D)),
                  blk((R, D))] + [full(w) for w in weights],
        out_specs=pl.BlockSpec((2, R, D), lambda i: (0, i, 0)),
        out_shape=jax.ShapeDtypeStruct((2, N, D), jnp.float32),
    )(x, sum_s, cnt_s, sum_a, cnt_a, *weights)


def _tc_b_body(h1_ref, ss_ref, cs_ref, sa_ref, ca_ref,
               WsT_ref, WnT_ref, b0s_ref, gs_ref, bs_ref,
               WaT_ref, WanT_ref, b0a_ref, ga_ref, ba_ref,
               ncg_ref, ncb_ref, fg1t_ref, fg1b_ref, fg2w_ref, fg2b_ref,
               rpt_ref, rpb_ref, out_ref):
    hs1 = h1_ref[0]
    ha1 = h1_ref[1]
    nei_s = ss_ref[...] / (cs_ref[:, 0:1] + 1e-12)
    nei_a = sa_ref[...] / (ca_ref[:, 0:1] + 1e-12)
    hs2 = _sage_post(hs1, nei_s, WsT_ref[...], WnT_ref[...],
                     b0s_ref[...], gs_ref[...], bs_ref[...])
    ha2 = _sage_post(ha1, nei_a, WaT_ref[...], WanT_ref[...],
                     b0a_ref[...], ga_ref[...], ba_ref[...])

    pieces = (hs1, hs2, ha1, ha2)
    tot = sum(jnp.sum(p, axis=-1, keepdims=True) for p in pieces)
    totq = sum(jnp.sum(p * p, axis=-1, keepdims=True) for p in pieces)
    mu = tot * (1.0 / 512.0)
    var = totq * (1.0 / 512.0) - mu * mu
    rstd = lax.rsqrt(var + 1e-5)
    ncg = ncg_ref[...]
    ncb = ncb_ref[...]
    fg1t = fg1t_ref[...]
    acc = fg1b_ref[...]
    for i, p in enumerate(pieces):
        cc = (p - mu) * rstd * ncg[i] + ncb[i]
        acc = acc + jnp.dot(cc, fg1t[i * D:(i + 1) * D],
                            preferred_element_type=jnp.float32)
    g1 = jnp.maximum(acc, 0.0)
    w = jax.nn.sigmoid(jnp.sum(g1 * fg2w_ref[...], axis=-1, keepdims=True)
                       + fg2b_ref[0, 0])
    f1 = w * hs1 + (1.0 - w) * ha1
    f2 = w * hs2 + (1.0 - w) * ha2
    rpt = rpt_ref[...]
    out_ref[...] = (jnp.dot(f1, rpt[0:D], preferred_element_type=jnp.float32)
                    + jnp.dot(f2, rpt[D:2 * D],
                              preferred_element_type=jnp.float32)
                    + rpb_ref[...])


def _tc_b(h1, sum_s, cnt_s, sum_a, cnt_a, *weights):
    blk = lambda shp: pl.BlockSpec(shp, lambda i: (i, 0))
    full = lambda a: pl.BlockSpec(a.shape, lambda i: (0,) * a.ndim)
    return pl.pallas_call(
        _tc_b_body,
        grid=(N // R,),
        in_specs=[pl.BlockSpec((2, R, D), lambda i: (0, i, 0)),
                  blk((R, D)), blk((R, D)), blk((R, D)), blk((R, D))]
                 + [full(w) for w in weights],
        out_specs=blk((R, D)),
        out_shape=jax.ShapeDtypeStruct((N, D), jnp.float32),
    )(h1, sum_s, cnt_s, sum_a, cnt_a, *weights)


def kernel(x, edge_spatial, edge_attr,
           s0_Wself, s0_bself, s0_Wnei, s0_bnei, s0_g, s0_b,
           s1_Wself, s1_bself, s1_Wnei, s1_bnei, s1_g, s1_b,
           a0_Wself, a0_bself, a0_Wnei, a0_bnei, a0_g, a0_b,
           a1_Wself, a1_bself, a1_Wnei, a1_bnei, a1_g, a1_b,
           nc_g, nc_b, fg1_W, fg1_b, fg2_W, fg2_b, rp_W, rp_b):
    f32 = jnp.float32
    pad_r = jnp.full((EPAD - E,), NP - 8, jnp.int32)
    pad_c = jnp.zeros((EPAD - E,), jnp.int32)
    pad = lambda v, p: jnp.concatenate([v, p])
    idx4 = lambda a, b: jnp.stack([a, b]).reshape(2, NS * NOUT, SUP, K)
    rows_all = idx4(pad(edge_spatial[0], pad_r), pad(edge_attr[0], pad_r))
    cols1 = idx4(pad(edge_spatial[1], pad_c), pad(edge_attr[1], pad_c))
    cols2 = idx4(pad(edge_spatial[1], pad_c), pad(edge_attr[1] + N, pad_c))
    z128 = jnp.zeros((K, D), f32)
    ones_h = jnp.ones((K, D), f32)

    sums0, counts = _sc_phase1(x, rows_all, cols1, z128, ones_h)
    sum_s0, sum_a0 = sums0[0], sums0[1]
    cnt_s, cnt_a = counts[0], counts[1]

    row = lambda v: v.reshape(1, -1)
    wa = (s0_Wself.T, s0_Wnei.T, row(s0_bself + s0_bnei), row(s0_g),
          row(s0_b), a0_Wself.T, a0_Wnei.T, row(a0_bself + a0_bnei),
          row(a0_g), row(a0_b))
    h1 = _tc_a(x, sum_s0, cnt_s, sum_a0, cnt_a, *wa)

    sums1, = _sc_phase2(h1.reshape(2 * N, D), rows_all, cols2, z128, ones_h)
    sum_s1, sum_a1 = sums1[0], sums1[1]

    wb = (s1_Wself.T, s1_Wnei.T, row(s1_bself + s1_bnei), row(s1_g),
          row(s1_b), a1_Wself.T, a1_Wnei.T, row(a1_bself + a1_bnei),
          row(a1_g), row(a1_b),
          nc_g.reshape(4, D), nc_b.reshape(4, D),
          fg1_W.T, row(fg1_b), fg2_W, fg2_b.reshape(1, 1),
          rp_W.T, rp_b.reshape(1, D))
    return _tc_b(h1, sum_s1, cnt_s, sum_a1, cnt_a, *wb)


# speedup vs baseline: 2.3772x; 1.0072x over previous
import functools

import jax
import jax.numpy as jnp
from jax import lax
from jax.experimental import pallas as pl
from jax.experimental.pallas import tpu as pltpu
from jax.experimental.pallas import tpu_sc as plsc

N = 10000
E = 320000
D = 128
NS = 16
K = 80
EPT = 20000
NCHUNK = EPT // K
SUP = 10
NOUT = NCHUNK // SUP
EPAD = NS * EPT
NP = 10240
RPW = NP // NS

_mesh = plsc.VectorSubcoreMesh(core_axis_name="c", subcore_axis_name="s",
                               num_cores=2, num_subcores=NS)


def _make_sc_scatter(with_counts: bool):
    out_type = [
        jax.ShapeDtypeStruct((2, NP, D), jnp.float32),
    ]
    if with_counts:
        out_type += [
            jax.ShapeDtypeStruct((2, NP, D), jnp.float32),
        ]
    scratch = [
        pltpu.VMEM_SHARED((NP, D), jnp.float32),
        pltpu.VMEM((SUP, K), jnp.int32),
        pltpu.VMEM((SUP, K), jnp.int32),
        pltpu.VMEM((K, D), jnp.float32),
        pltpu.VMEM((K, D), jnp.float32),
        pltpu.SemaphoreType.DMA,
        pltpu.SemaphoreType.DMA,
        pltpu.SemaphoreType.DMA,
        pltpu.SemaphoreType.DMA,
    ]

    @functools.partial(pl.kernel, out_type=out_type, mesh=_mesh,
                       scratch_types=scratch)
    def sc_kernel(table, rows_all, cols_all, z128, ones_h, *rest):
        if with_counts:
            sums, counts, acc, idx_row, idx_col, bufa, bufb, gsa, gsb, ssa, ssb = rest
        else:
            sums, acc, idx_row, idx_col, bufa, bufb, gsa, gsb, ssa, ssb = rest
        bufs = (bufa, bufb)
        gsems = (gsa, gsb)
        ssems = (ssa, ssb)

        c = lax.axis_index("c")
        s = lax.axis_index("s")

        def tile_slices():
            return [pl.ds(s * RPW + i * K, K) for i in range(RPW // K)]

        def zero_acc():
            pltpu.sync_copy(z128, bufa)
            for sli in tile_slices():
                pltpu.sync_copy(bufa, acc.at[sli])

        def publish(dst):
            for sli in tile_slices():
                pltpu.sync_copy(acc.at[sli], bufa)
                pltpu.sync_copy(bufa, dst.at[c, sli])

        def stage_idx(jo, cols=True):
            pltpu.sync_copy(rows_all.at[c, s * NOUT + jo], idx_row)
            if cols:
                pltpu.sync_copy(cols_all.at[c, s * NOUT + jo], idx_col)

        zero_acc()

        if with_counts:
            pltpu.sync_copy(ones_h, bufb)
            plsc.subcore_barrier()

            def outer0(jo, carry):
                stage_idx(jo, cols=False)
                cps = [pltpu.async_copy(bufb, acc.at[idx_row.at[j]],
                                        ssems[j % 2], add=True)
                       for j in range(SUP)]
                for cp in cps:
                    cp.wait()
                return carry

            lax.fori_loop(0, NOUT, outer0, 0)
            plsc.subcore_barrier()
            publish(counts)
            zero_acc()

        plsc.subcore_barrier()

        def outer(jo, carry):
            stage_idx(jo)
            gcp = [None] * SUP
            scp = [None] * SUP
            gcp[0] = pltpu.async_copy(table.at[idx_col.at[0]], bufs[0],
                                      gsems[0])
            for j in range(SUP):
                b = j % 2
                if j + 1 < SUP:
                    if j >= 1:
                        scp[j - 1].wait()
                    gcp[j + 1] = pltpu.async_copy(
                        table.at[idx_col.at[j + 1]], bufs[1 - b],
                        gsems[1 - b])
                gcp[j].wait()
                scp[j] = pltpu.async_copy(bufs[b], acc.at[idx_row.at[j]],
                                          ssems[b], add=True)
            scp[SUP - 2].wait()
            scp[SUP - 1].wait()
            return carry

        lax.fori_loop(0, NOUT, outer, 0)

        plsc.subcore_barrier()
        publish(sums)

    return sc_kernel


_sc_phase1 = _make_sc_scatter(with_counts=True)
_sc_phase2 = _make_sc_scatter(with_counts=False)

R = 1000


def _ln_gelu(h, g, b):
    mu = jnp.mean(h, axis=-1, keepdims=True)
    var = jnp.mean((h - mu) ** 2, axis=-1, keepdims=True)
    y = (h - mu) * lax.rsqrt(var + 1e-5) * g + b
    return 0.5 * y * (1.0 + lax.erf(y * 0.7071067811865476))


def _sage_post(x, nei, WsT, WnT, b0, g, b):
    h = (jnp.dot(x, WsT, preferred_element_type=jnp.float32)
         + jnp.dot(nei, WnT, preferred_element_type=jnp.float32) + b0)
    return _ln_gelu(h, g, b)


def _tc_a_body(x_ref, ss_ref, cs_ref, sa_ref, ca_ref,
               WsT_ref, WnT_ref, b0s_ref, gs_ref, bs_ref,
               WaT_ref, WanT_ref, b0a_ref, ga_ref, ba_ref, out_ref):
    x = x_ref[...]
    nei_s = ss_ref[...] / (cs_ref[:, 0:1] + 1e-12)
    nei_a = sa_ref[...] / (ca_ref[:, 0:1] + 1e-12)
    out_ref[0] = _sage_post(x, nei_s, WsT_ref[...], WnT_ref[...],
                            b0s_ref[...], gs_ref[...], bs_ref[...])
    out_ref[1] = _sage_post(x, nei_a, WaT_ref[...], WanT_ref[...],
                            b0a_ref[...], ga_ref[...], ba_ref[...])


def _tc_a(x, sum_s, cnt_s, sum_a, cnt_a, *weights):
    blk = lambda shp: pl.BlockSpec(shp, lambda i: (i, 0))
    full = lambda a: pl.BlockSpec(a.shape, lambda i: (0,) * a.ndim)
    return pl.pallas_call(
        _tc_a_body,
        grid=(N // R,),
        in_specs=[blk((R, D)), blk((R, D)), blk((R, D)), blk((R, D)),
                  blk((R, D))] + [full(w) for w in weights],
        out_specs=pl.BlockSpec((2, R, D), lambda i: (0, i, 0)),
        out_shape=jax.ShapeDtypeStruct((2, N, D), jnp.float32),
    )(x, sum_s, cnt_s, sum_a, cnt_a, *weights)


def _tc_b_body(h1_ref, ss_ref, cs_ref, sa_ref, ca_ref,
               WsT_ref, WnT_ref, b0s_ref, gs_ref, bs_ref,
               WaT_ref, WanT_ref, b0a_ref, ga_ref, ba_ref,
               ncg_ref, ncb_ref, fg1t_ref, fg1b_ref, fg2w_ref, fg2b_ref,
               rpt_ref, rpb_ref, out_ref):
    hs1 = h1_ref[0]
    ha1 = h1_ref[1]
    nei_s = ss_ref[...] / (cs_ref[:, 0:1] + 1e-12)
    nei_a = sa_ref[...] / (ca_ref[:, 0:1] + 1e-12)
    hs2 = _sage_post(hs1, nei_s, WsT_ref[...], WnT_ref[...],
                     b0s_ref[...], gs_ref[...], bs_ref[...])
    ha2 = _sage_post(ha1, nei_a, WaT_ref[...], WanT_ref[...],
                     b0a_ref[...], ga_ref[...], ba_ref[...])

    pieces = (hs1, hs2, ha1, ha2)
    tot = sum(jnp.sum(p, axis=-1, keepdims=True) for p in pieces)
    totq = sum(jnp.sum(p * p, axis=-1, keepdims=True) for p in pieces)
    mu = tot * (1.0 / 512.0)
    var = totq * (1.0 / 512.0) - mu * mu
    rstd = lax.rsqrt(var + 1e-5)
    ncg = ncg_ref[...]
    ncb = ncb_ref[...]
    fg1t = fg1t_ref[...]
    acc = fg1b_ref[...]
    for i, p in enumerate(pieces):
        cc = (p - mu) * rstd * ncg[i] + ncb[i]
        acc = acc + jnp.dot(cc, fg1t[i * D:(i + 1) * D],
                            preferred_element_type=jnp.float32)
    g1 = jnp.maximum(acc, 0.0)
    w = jax.nn.sigmoid(jnp.sum(g1 * fg2w_ref[...], axis=-1, keepdims=True)
                       + fg2b_ref[0, 0])
    f1 = w * hs1 + (1.0 - w) * ha1
    f2 = w * hs2 + (1.0 - w) * ha2
    rpt = rpt_ref[...]
    out_ref[...] = (jnp.dot(f1, rpt[0:D], preferred_element_type=jnp.float32)
                    + jnp.dot(f2, rpt[D:2 * D],
                              preferred_element_type=jnp.float32)
                    + rpb_ref[...])


def _tc_b(h1, sum_s, cnt_s, sum_a, cnt_a, *weights):
    blk = lambda shp: pl.BlockSpec(shp, lambda i: (i, 0))
    full = lambda a: pl.BlockSpec(a.shape, lambda i: (0,) * a.ndim)
    return pl.pallas_call(
        _tc_b_body,
        grid=(N // R,),
        in_specs=[pl.BlockSpec((2, R, D), lambda i: (0, i, 0)),
                  blk((R, D)), blk((R, D)), blk((R, D)), blk((R, D))]
                 + [full(w) for w in weights],
        out_specs=blk((R, D)),
        out_shape=jax.ShapeDtypeStruct((N, D), jnp.float32),
    )(h1, sum_s, cnt_s, sum_a, cnt_a, *weights)


def kernel(x, edge_spatial, edge_attr,
           s0_Wself, s0_bself, s0_Wnei, s0_bnei, s0_g, s0_b,
           s1_Wself, s1_bself, s1_Wnei, s1_bnei, s1_g, s1_b,
           a0_Wself, a0_bself, a0_Wnei, a0_bnei, a0_g, a0_b,
           a1_Wself, a1_bself, a1_Wnei, a1_bnei, a1_g, a1_b,
           nc_g, nc_b, fg1_W, fg1_b, fg2_W, fg2_b, rp_W, rp_b):
    f32 = jnp.float32
    pad_r = jnp.full((EPAD - E,), NP - 8, jnp.int32)
    pad_c = jnp.zeros((EPAD - E,), jnp.int32)
    pad = lambda v, p: jnp.concatenate([v, p])
    idx4 = lambda a, b: jnp.stack([a, b]).reshape(2, NS * NOUT, SUP, K)
    rows_all = idx4(pad(edge_spatial[0], pad_r), pad(edge_attr[0], pad_r))
    cols1 = idx4(pad(edge_spatial[1], pad_c), pad(edge_attr[1], pad_c))
    cols2 = idx4(pad(edge_spatial[1], pad_c), pad(edge_attr[1] + N, pad_c))
    z128 = jnp.zeros((K, D), f32)
    ones_h = jnp.ones((K, D), f32)

    sums0, counts = _sc_phase1(x, rows_all, cols1, z128, ones_h)
    sum_s0, sum_a0 = sums0[0], sums0[1]
    cnt_s, cnt_a = counts[0], counts[1]

    row = lambda v: v.reshape(1, -1)
    wa = (s0_Wself.T, s0_Wnei.T, row(s0_bself + s0_bnei), row(s0_g),
          row(s0_b), a0_Wself.T, a0_Wnei.T, row(a0_bself + a0_bnei),
          row(a0_g), row(a0_b))
    h1 = _tc_a(x, sum_s0, cnt_s, sum_a0, cnt_a, *wa)

    sums1, = _sc_phase2(h1.reshape(2 * N, D), rows_all, cols2, z128, ones_h)
    sum_s1, sum_a1 = sums1[0], sums1[1]

    wb = (s1_Wself.T, s1_Wnei.T, row(s1_bself + s1_bnei), row(s1_g),
          row(s1_b), a1_Wself.T, a1_Wnei.T, row(a1_bself + a1_bnei),
          row(a1_g), row(a1_b),
          nc_g.reshape(4, D), nc_b.reshape(4, D),
          fg1_W.T, row(fg1_b), fg2_W, fg2_b.reshape(1, 1),
          rp_W.T, rp_b.reshape(1, D))
    return _tc_b(h1, sum_s1, cnt_s, sum_a1, cnt_a, *wb)


# speedup vs baseline: 2.7195x; 1.1440x over previous
import functools

import jax
import jax.numpy as jnp
from jax import lax
from jax.experimental import pallas as pl
from jax.experimental.pallas import tpu as pltpu
from jax.experimental.pallas import tpu_sc as plsc

N = 10000
E = 320000
D = 128
NS = 16
K = 80
EPT = 20000
NCHUNK = EPT // K
SUP = 10
NOUT = NCHUNK // SUP
EPAD = NS * EPT
NP = 10240
RPW = NP // NS

_mesh = plsc.VectorSubcoreMesh(core_axis_name="c", subcore_axis_name="s",
                               num_cores=2, num_subcores=NS)


def _make_sc_scatter(with_counts: bool):
    out_type = [
        jax.ShapeDtypeStruct((2, NP, D), jnp.float32),
    ]
    if with_counts:
        out_type += [
            jax.ShapeDtypeStruct((2, NP // D, D), jnp.float32),
        ]
    scratch = [
        pltpu.VMEM_SHARED((NP, D), jnp.float32),
        pltpu.VMEM((SUP, K), jnp.int32),
        pltpu.VMEM((SUP, K), jnp.int32),
        pltpu.VMEM((K, D), jnp.float32),
        pltpu.VMEM((K, D), jnp.float32),
        pltpu.SemaphoreType.DMA,
        pltpu.SemaphoreType.DMA,
        pltpu.SemaphoreType.DMA,
        pltpu.SemaphoreType.DMA,
    ]
    if with_counts:
        scratch += [
            pltpu.VMEM((NP // D, D), jnp.float32),
            pltpu.VMEM((NP // D,), jnp.int32),
            pltpu.VMEM_SHARED((NP // D, D), jnp.float32),
        ]

    @functools.partial(
        pl.kernel, out_type=out_type, mesh=_mesh, scratch_types=scratch,
        compiler_params=pltpu.CompilerParams(needs_layout_passes=False))
    def sc_kernel(table, rows_all, cols_all, z128, arange_h, *rest):
        if with_counts:
            (sums, counts, acc, idx_row, idx_col, bufa, bufb,
             gsa, gsb, ssa, ssb, hist, viota, hist_sp) = rest
        else:
            sums, acc, idx_row, idx_col, bufa, bufb, gsa, gsb, ssa, ssb = rest
        bufs = (bufa, bufb)
        gsems = (gsa, gsb)
        ssems = (ssa, ssb)

        c = lax.axis_index("c")
        s = lax.axis_index("s")

        def tile_slices():
            return [pl.ds(s * RPW + i * K, K) for i in range(RPW // K)]

        def zero_acc():
            pltpu.sync_copy(z128, bufa)
            for sli in tile_slices():
                pltpu.sync_copy(bufa, acc.at[sli])

        def publish(dst):
            for sli in tile_slices():
                pltpu.sync_copy(acc.at[sli], bufa)
                pltpu.sync_copy(bufa, dst.at[c, sli])

        def stage_idx(jo, cols=True):
            pltpu.sync_copy(rows_all.at[c, s * NOUT + jo], idx_row)
            if cols:
                pltpu.sync_copy(cols_all.at[c, s * NOUT + jo], idx_col)

        zero_acc()

        if with_counts:
            pltpu.sync_copy(z128, hist)
            pltpu.sync_copy(arange_h, viota)

            @pl.when(s == 0)
            def _():
                pltpu.sync_copy(z128, bufb)
                pltpu.sync_copy(bufb, hist_sp)

            ones16 = jnp.full((16,), 1.0, jnp.float32)

            def outer0(jo, carry):
                stage_idx(jo, cols=False)
                for j in range(SUP):
                    for g in range(K // 16):
                        v = idx_row[j, pl.ds(g * 16, 16)]
                        plsc.addupdate_scatter(
                            hist,
                            [lax.shift_right_logical(v, 7),
                             lax.bitwise_and(v, 127)],
                            ones16)
                return carry

            lax.fori_loop(0, NOUT, outer0, 0)
            plsc.subcore_barrier()
            pltpu.sync_copy(hist, hist_sp.at[viota], add=True)
            plsc.subcore_barrier()

            @pl.when(s < (NP // D) // 8)
            def _():
                sl8 = pl.ds(s * 8, 8)
                pltpu.sync_copy(hist_sp.at[sl8], bufb.at[pl.ds(0, 8)])
                pltpu.sync_copy(bufb.at[pl.ds(0, 8)], counts.at[c, sl8])

        plsc.subcore_barrier()

        def outer(jo, carry):
            stage_idx(jo)
            gcp = [None] * SUP
            scp = [None] * SUP
            gcp[0] = pltpu.async_copy(table.at[idx_col.at[0]], bufs[0],
                                      gsems[0])
            for j in range(SUP):
                b = j % 2
                if j + 1 < SUP:
                    if j >= 1:
                        scp[j - 1].wait()
                    gcp[j + 1] = pltpu.async_copy(
                        table.at[idx_col.at[j + 1]], bufs[1 - b],
                        gsems[1 - b])
                gcp[j].wait()
                scp[j] = pltpu.async_copy(bufs[b], acc.at[idx_row.at[j]],
                                          ssems[b], add=True)
            scp[SUP - 2].wait()
            scp[SUP - 1].wait()
            return carry

        lax.fori_loop(0, NOUT, outer, 0)

        plsc.subcore_barrier()
        publish(sums)

    return sc_kernel


_sc_phase1 = _make_sc_scatter(with_counts=True)
_sc_phase2 = _make_sc_scatter(with_counts=False)

R = 1000


def _ln_gelu(h, g, b):
    mu = jnp.mean(h, axis=-1, keepdims=True)
    var = jnp.mean((h - mu) ** 2, axis=-1, keepdims=True)
    y = (h - mu) * lax.rsqrt(var + 1e-5) * g + b
    return 0.5 * y * (1.0 + lax.erf(y * 0.7071067811865476))


def _sage_post(x, nei, WsT, WnT, b0, g, b):
    h = (jnp.dot(x, WsT, preferred_element_type=jnp.float32)
         + jnp.dot(nei, WnT, preferred_element_type=jnp.float32) + b0)
    return _ln_gelu(h, g, b)


def _tc_a_body(x_ref, ss_ref, cs_ref, sa_ref, ca_ref,
               WsT_ref, WnT_ref, b0s_ref, gs_ref, bs_ref,
               WaT_ref, WanT_ref, b0a_ref, ga_ref, ba_ref, out_ref):
    x = x_ref[...]
    nei_s = ss_ref[...] / (cs_ref[...] + 1e-12)
    nei_a = sa_ref[...] / (ca_ref[...] + 1e-12)
    out_ref[0] = _sage_post(x, nei_s, WsT_ref[...], WnT_ref[...],
                            b0s_ref[...], gs_ref[...], bs_ref[...])
    out_ref[1] = _sage_post(x, nei_a, WaT_ref[...], WanT_ref[...],
                            b0a_ref[...], ga_ref[...], ba_ref[...])


def _tc_a(x, sum_s, cnt_s, sum_a, cnt_a, *weights):
    blk = lambda shp: pl.BlockSpec(shp, lambda i: (i, 0))
    full = lambda a: pl.BlockSpec(a.shape, lambda i: (0,) * a.ndim)
    return pl.pallas_call(
        _tc_a_body,
        grid=(N // R,),
        in_specs=[blk((R, D)), blk((R, D)), blk((R, 1)), blk((R, D)),
                  blk((R, 1))] + [full(w) for w in weights],
        out_specs=pl.BlockSpec((2, R, D), lambda i: (0, i, 0)),
        out_shape=jax.ShapeDtypeStruct((2, N, D), jnp.float32),
    )(x, sum_s, cnt_s, sum_a, cnt_a, *weights)


def _tc_b_body(h1_ref, ss_ref, cs_ref, sa_ref, ca_ref,
               WsT_ref, WnT_ref, b0s_ref, gs_ref, bs_ref,
               WaT_ref, WanT_ref, b0a_ref, ga_ref, ba_ref,
               ncg_ref, ncb_ref, fg1t_ref, fg1b_ref, fg2w_ref, fg2b_ref,
               rpt_ref, rpb_ref, out_ref):
    hs1 = h1_ref[0]
    ha1 = h1_ref[1]
    nei_s = ss_ref[...] / (cs_ref[...] + 1e-12)
    nei_a = sa_ref[...] / (ca_ref[...] + 1e-12)
    hs2 = _sage_post(hs1, nei_s, WsT_ref[...], WnT_ref[...],
                     b0s_ref[...], gs_ref[...], bs_ref[...])
    ha2 = _sage_post(ha1, nei_a, WaT_ref[...], WanT_ref[...],
                     b0a_ref[...], ga_ref[...], ba_ref[...])

    pieces = (hs1, hs2, ha1, ha2)
    tot = sum(jnp.sum(p, axis=-1, keepdims=True) for p in pieces)
    totq = sum(jnp.sum(p * p, axis=-1, keepdims=True) for p in pieces)
    mu = tot * (1.0 / 512.0)
    var = totq * (1.0 / 512.0) - mu * mu
    rstd = lax.rsqrt(var + 1e-5)
    ncg = ncg_ref[...]
    ncb = ncb_ref[...]
    fg1t = fg1t_ref[...]
    acc = fg1b_ref[...]
    for i, p in enumerate(pieces):
        cc = (p - mu) * rstd * ncg[i] + ncb[i]
        acc = acc + jnp.dot(cc, fg1t[i * D:(i + 1) * D],
                            preferred_element_type=jnp.float32)
    g1 = jnp.maximum(acc, 0.0)
    w = jax.nn.sigmoid(jnp.sum(g1 * fg2w_ref[...], axis=-1, keepdims=True)
                       + fg2b_ref[0, 0])
    f1 = w * hs1 + (1.0 - w) * ha1
    f2 = w * hs2 + (1.0 - w) * ha2
    rpt = rpt_ref[...]
    out_ref[...] = (jnp.dot(f1, rpt[0:D], preferred_element_type=jnp.float32)
                    + jnp.dot(f2, rpt[D:2 * D],
                              preferred_element_type=jnp.float32)
                    + rpb_ref[...])


def _tc_b(h1, sum_s, cnt_s, sum_a, cnt_a, *weights):
    blk = lambda shp: pl.BlockSpec(shp, lambda i: (i, 0))
    full = lambda a: pl.BlockSpec(a.shape, lambda i: (0,) * a.ndim)
    return pl.pallas_call(
        _tc_b_body,
        grid=(N // R,),
        in_specs=[pl.BlockSpec((2, R, D), lambda i: (0, i, 0)),
                  blk((R, D)), blk((R, 1)), blk((R, D)), blk((R, 1))]
                 + [full(w) for w in weights],
        out_specs=blk((R, D)),
        out_shape=jax.ShapeDtypeStruct((N, D), jnp.float32),
    )(h1, sum_s, cnt_s, sum_a, cnt_a, *weights)


def kernel(x, edge_spatial, edge_attr,
           s0_Wself, s0_bself, s0_Wnei, s0_bnei, s0_g, s0_b,
           s1_Wself, s1_bself, s1_Wnei, s1_bnei, s1_g, s1_b,
           a0_Wself, a0_bself, a0_Wnei, a0_bnei, a0_g, a0_b,
           a1_Wself, a1_bself, a1_Wnei, a1_bnei, a1_g, a1_b,
           nc_g, nc_b, fg1_W, fg1_b, fg2_W, fg2_b, rp_W, rp_b):
    f32 = jnp.float32
    pad_r = jnp.full((EPAD - E,), NP - 8, jnp.int32)
    pad_c = jnp.zeros((EPAD - E,), jnp.int32)
    pad = lambda v, p: jnp.concatenate([v, p])
    idx4 = lambda a, b: jnp.stack([a, b]).reshape(2, NS * NOUT, SUP, K)
    rows_all = idx4(pad(edge_spatial[0], pad_r), pad(edge_attr[0], pad_r))
    cols1 = idx4(pad(edge_spatial[1], pad_c), pad(edge_attr[1], pad_c))
    cols2 = idx4(pad(edge_spatial[1], pad_c), pad(edge_attr[1] + N, pad_c))
    z128 = jnp.zeros((K, D), f32)
    arange_h = jnp.arange(NP // D, dtype=jnp.int32)

    sums0, counts = _sc_phase1(x, rows_all, cols1, z128, arange_h)
    sum_s0, sum_a0 = sums0[0], sums0[1]
    cnt_s = counts[0].reshape(NP, 1)[:N]
    cnt_a = counts[1].reshape(NP, 1)[:N]

    row = lambda v: v.reshape(1, -1)
    wa = (s0_Wself.T, s0_Wnei.T, row(s0_bself + s0_bnei), row(s0_g),
          row(s0_b), a0_Wself.T, a0_Wnei.T, row(a0_bself + a0_bnei),
          row(a0_g), row(a0_b))
    h1 = _tc_a(x, sum_s0, cnt_s, sum_a0, cnt_a, *wa)

    sums1, = _sc_phase2(h1.reshape(2 * N, D), rows_all, cols2, z128,
                        arange_h)
    sum_s1, sum_a1 = sums1[0], sums1[1]

    wb = (s1_Wself.T, s1_Wnei.T, row(s1_bself + s1_bnei), row(s1_g),
          row(s1_b), a1_Wself.T, a1_Wnei.T, row(a1_bself + a1_bnei),
          row(a1_g), row(a1_b),
          nc_g.reshape(4, D), nc_b.reshape(4, D),
          fg1_W.T, row(fg1_b), fg2_W, fg2_b.reshape(1, 1),
          rp_W.T, rp_b.reshape(1, D))
    return _tc_b(h1, sum_s1, cnt_s, sum_a1, cnt_a, *wb)
